# baseline (speedup 1.0000x reference)
import functools

import jax
import jax.numpy as jnp
from jax import lax
from jax.experimental import pallas as pl
from jax.experimental.pallas import tpu as pltpu
from jax.experimental.pallas import tpu_sc as plsc

_LANES = 16
_CCHUNK = 128


def _sc_workers():
    try:
        info = plsc.get_sparse_core_info()
        return info.num_cores, info.num_subcores
    except Exception:
        return 2, 16


def _tc_pack(w2d):
    H, L = w2d.shape

    def body(w_ref, o_ref):
        t = 2.0 / (1.0 + jnp.exp(-w_ref[...]))
        x3 = t.reshape(H // 2, 2, L)
        lo = jax.lax.bitcast_convert_type(
            x3[:, 0, :].astype(jnp.bfloat16), jnp.uint16).astype(jnp.uint32)
        hi = jax.lax.bitcast_convert_type(
            x3[:, 1, :].astype(jnp.bfloat16), jnp.uint16).astype(jnp.uint32)
        word = jnp.bitwise_or(lo, jax.lax.shift_left(hi, jnp.uint32(16)))
        o_ref[...] = jax.lax.bitcast_convert_type(word, jnp.int32)

    return pl.pallas_call(
        body,
        out_shape=jax.ShapeDtypeStruct((H // 2, L), jnp.int32),
    )(w2d)


def kernel(r_ids, W):
    B, F = r_ids.shape
    V = W.shape[0]

    idx_t = r_ids.astype(jnp.int32).T
    Vp = -(-V // 256) * 256
    w_pad = jnp.pad(W, ((0, Vp - V), (0, 0)))
    w_packed = _tc_pack(w_pad.reshape(Vp // 128, 128))

    NC, NS = _sc_workers()
    NW = NC * NS
    cols_per_w = B // NW
    assert cols_per_w * NW == B
    n_chunks = cols_per_w // _CCHUNK
    assert n_chunks * _CCHUNK == cols_per_w

    mesh = plsc.VectorSubcoreMesh(
        core_axis_name="c", subcore_axis_name="s",
        num_cores=NC, num_subcores=NS,
    )

    @functools.partial(
        pl.kernel,
        out_type=jax.ShapeDtypeStruct((F, B), jnp.float32),
        mesh=mesh,
        compiler_params=pltpu.CompilerParams(needs_layout_passes=False),
        scratch_types=[
            pltpu.VMEM((Vp // 256, 128), jnp.int32),
            pltpu.VMEM((F, _CCHUNK), jnp.int32),
            pltpu.VMEM((F, _CCHUNK), jnp.int32),
            pltpu.VMEM((F, _CCHUNK), jnp.int32),
            pltpu.VMEM((F, _CCHUNK), jnp.float32),
            pltpu.VMEM((F, _CCHUNK), jnp.float32),
            pltpu.VMEM((F, _CCHUNK), jnp.float32),
            pltpu.SemaphoreType.DMA,
            pltpu.SemaphoreType.DMA,
            pltpu.SemaphoreType.DMA,
            pltpu.SemaphoreType.DMA,
            pltpu.SemaphoreType.DMA,
            pltpu.SemaphoreType.DMA,
            pltpu.SemaphoreType.DMA,
        ],
    )
    def sc_gather(idx_hbm, tab_hbm, out_hbm, tab_v, idx_a, idx_b, idx_c,
                  out_a, out_b, out_c, sem_t, sem_ia, sem_ib, sem_ic,
                  sem_oa, sem_ob, sem_oc):
        wid = lax.axis_index("s") * NC + lax.axis_index("c")
        col0 = wid * cols_per_w

        nbuf = 3
        idx_bufs = (idx_a, idx_b, idx_c)
        out_bufs = (out_a, out_b, out_c)
        idx_sems = (sem_ia, sem_ib, sem_ic)
        out_sems = (sem_oa, sem_ob, sem_oc)

        tab_cp = pltpu.async_copy(tab_hbm, tab_v, sem_t)
        idx_cps = [None] * n_chunks
        out_cps = [None] * n_chunks
        for k in range(min(nbuf, n_chunks)):
            idx_cps[k] = pltpu.async_copy(
                idx_hbm.at[:, pl.ds(col0 + k * _CCHUNK, _CCHUNK)],
                idx_bufs[k], idx_sems[k])
        tab_cp.wait()
        n_vecs = _CCHUNK // _LANES

        for k in range(n_chunks):
            p = k % nbuf
            idx_cps[k].wait()
            if k >= nbuf:
                out_cps[k - nbuf].wait()
            ib, ob = idx_bufs[p], out_bufs[p]

            @plsc.parallel_loop(0, F, unroll=8)
            def gather_body(r, ib=ib, ob=ob):
                for c in range(n_vecs):
                    iv = ib[r, pl.ds(c * _LANES, _LANES)]
                    wr = lax.shift_right_logical(iv, 8)
                    wc = jnp.bitwise_and(iv, 127)
                    word = plsc.load_gather(tab_v, [wr, wc])
                    half = jnp.bitwise_and(lax.shift_right_logical(iv, 7), 1)
                    sh = lax.shift_left(half, 4)
                    bits = lax.shift_left(
                        lax.shift_right_logical(word, sh), 16)
                    ob[r, pl.ds(c * _LANES, _LANES)] = plsc.bitcast(
                        bits, jnp.float32)

            out_cps[k] = pltpu.async_copy(
                ob, out_hbm.at[:, pl.ds(col0 + k * _CCHUNK, _CCHUNK)],
                out_sems[p])
            if k + nbuf < n_chunks:
                idx_cps[k + nbuf] = pltpu.async_copy(
                    idx_hbm.at[:, pl.ds(col0 + (k + nbuf) * _CCHUNK, _CCHUNK)],
                    idx_bufs[p], idx_sems[p])

        for k in range(max(0, n_chunks - nbuf), n_chunks):
            out_cps[k].wait()

    out_t = sc_gather(idx_t, w_packed)
    return out_t.T


# speedup vs baseline: 1.0476x; 1.0476x over previous
import functools

import jax
import jax.numpy as jnp
from jax import lax
from jax.experimental import pallas as pl
from jax.experimental.pallas import tpu as pltpu
from jax.experimental.pallas import tpu_sc as plsc

_LANES = 16
_CCHUNK = 128


def _sc_workers():
    try:
        info = plsc.get_sparse_core_info()
        return info.num_cores, info.num_subcores
    except Exception:
        return 2, 16


def _tc_pack(w2d):
    H, L = w2d.shape

    def body(w_ref, o_ref):
        t = 2.0 / (1.0 + jnp.exp(-w_ref[...]))
        x3 = t.reshape(H // 2, 2, L)
        lo = jax.lax.bitcast_convert_type(
            x3[:, 0, :].astype(jnp.bfloat16), jnp.uint16).astype(jnp.uint32)
        hi = jax.lax.bitcast_convert_type(
            x3[:, 1, :].astype(jnp.bfloat16), jnp.uint16).astype(jnp.uint32)
        word = jnp.bitwise_or(lo, jax.lax.shift_left(hi, jnp.uint32(16)))
        o_ref[...] = jax.lax.bitcast_convert_type(word, jnp.int32)

    return pl.pallas_call(
        body,
        out_shape=jax.ShapeDtypeStruct((H // 2, L), jnp.int32),
    )(w2d)


def kernel(r_ids, W):
    B, F = r_ids.shape
    V = W.shape[0]

    idx_t = r_ids.astype(jnp.int32).T
    Vp = -(-V // 256) * 256
    w_pad = jnp.pad(W, ((0, Vp - V), (0, 0)))
    w_packed = _tc_pack(w_pad.reshape(Vp // 128, 128))

    NC, NS = _sc_workers()
    NW = NC * NS
    cols_per_w = B // NW
    assert cols_per_w * NW == B
    n_chunks = cols_per_w // _CCHUNK
    assert n_chunks * _CCHUNK == cols_per_w

    mesh = plsc.VectorSubcoreMesh(
        core_axis_name="c", subcore_axis_name="s",
        num_cores=NC, num_subcores=NS,
    )

    @functools.partial(
        pl.kernel,
        out_type=jax.ShapeDtypeStruct((F, B), jnp.float32),
        mesh=mesh,
        compiler_params=pltpu.CompilerParams(needs_layout_passes=False),
        scratch_types=[
            pltpu.VMEM((Vp // 256, 128), jnp.int32),
            pltpu.VMEM((F, _CCHUNK), jnp.int32),
            pltpu.VMEM((F, _CCHUNK), jnp.int32),
            pltpu.VMEM((F, _CCHUNK), jnp.int32),
            pltpu.VMEM((F, _CCHUNK), jnp.float32),
            pltpu.VMEM((F, _CCHUNK), jnp.float32),
            pltpu.VMEM((F, _CCHUNK), jnp.float32),
            pltpu.SemaphoreType.DMA,
            pltpu.SemaphoreType.DMA,
            pltpu.SemaphoreType.DMA,
            pltpu.SemaphoreType.DMA,
            pltpu.SemaphoreType.DMA,
            pltpu.SemaphoreType.DMA,
            pltpu.SemaphoreType.DMA,
        ],
    )
    def sc_gather(idx_hbm, tab_hbm, out_hbm, tab_v, idx_a, idx_b, idx_c,
                  out_a, out_b, out_c, sem_t, sem_ia, sem_ib, sem_ic,
                  sem_oa, sem_ob, sem_oc):
        wid = lax.axis_index("s") * NC + lax.axis_index("c")
        col0 = wid * cols_per_w

        nbuf = 3
        idx_bufs = (idx_a, idx_b, idx_c)
        out_bufs = (out_a, out_b, out_c)
        idx_sems = (sem_ia, sem_ib, sem_ic)
        out_sems = (sem_oa, sem_ob, sem_oc)

        tab_cp = pltpu.async_copy(tab_hbm, tab_v, sem_t)
        idx_cps = [None] * n_chunks
        out_cps = [None] * n_chunks
        for k in range(min(nbuf, n_chunks)):
            idx_cps[k] = pltpu.async_copy(
                idx_hbm.at[:, pl.ds(col0 + k * _CCHUNK, _CCHUNK)],
                idx_bufs[k], idx_sems[k])
        tab_cp.wait()
        n_vecs = _CCHUNK // _LANES

        for k in range(n_chunks):
            p = k % nbuf
            idx_cps[k].wait()
            if k >= nbuf:
                out_cps[k - nbuf].wait()
            ib, ob = idx_bufs[p], out_bufs[p]

            @plsc.parallel_loop(0, F, unroll=4)
            def gather_body(r, ib=ib, ob=ob):
                for c in range(n_vecs):
                    iv = ib[r, pl.ds(c * _LANES, _LANES)]
                    wr = lax.shift_right_logical(iv, 8)
                    wc = jnp.bitwise_and(iv, 127)
                    word = plsc.load_gather(tab_v, [wr, wc])
                    half = jnp.bitwise_and(lax.shift_right_logical(iv, 7), 1)
                    sh = lax.shift_left(half, 4)
                    bits = lax.shift_left(
                        lax.shift_right_logical(word, sh), 16)
                    ob[r, pl.ds(c * _LANES, _LANES)] = plsc.bitcast(
                        bits, jnp.float32)

            out_cps[k] = pltpu.async_copy(
                ob, out_hbm.at[:, pl.ds(col0 + k * _CCHUNK, _CCHUNK)],
                out_sems[p])
            if k + nbuf < n_chunks:
                idx_cps[k + nbuf] = pltpu.async_copy(
                    idx_hbm.at[:, pl.ds(col0 + (k + nbuf) * _CCHUNK, _CCHUNK)],
                    idx_bufs[p], idx_sems[p])

        for k in range(max(0, n_chunks - nbuf), n_chunks):
            out_cps[k].wait()

    out_t = sc_gather(idx_t, w_packed)
    return out_t.T
